# baseline (speedup 1.0000x reference)
import functools

import jax
import jax.numpy as jnp
from jax import lax
from jax.experimental import pallas as pl
from jax.experimental.pallas import tpu as pltpu
from jax.experimental.pallas import tpu_sc as plsc

D = 128
C = 512
NW = 32
T_ROWS = 60


def _encoder_body(n_edges, ea_hbm, w0_hbm, w1_hbm, w2_hbm, out_hbm,
                  attr_v, out_v, w0_v, w1_v, w2_v, t_v):
    num_chunks = n_edges // C
    wid = lax.axis_index("s") * 2 + lax.axis_index("c")

    pltpu.sync_copy(w0_hbm, w0_v)
    pltpu.sync_copy(w1_hbm, w1_v)
    pltpu.sync_copy(w2_hbm, w2_v)

    def build(c60, carry):
        i0 = c60 // 12
        r = c60 - i0 * 12
        i1 = r // 2
        i2 = r - i1 * 2
        tb = pl.multiple_of(c60 * D, D)
        b0 = pl.multiple_of(i0 * D, D)
        b1 = pl.multiple_of(i1 * D, D)
        b2 = pl.multiple_of(i2 * D, D)
        for j in range(D // 16):
            o = j * 16
            t_v[pl.ds(tb + o, 16)] = (w0_v[pl.ds(b0 + o, 16)]
                                      + w1_v[pl.ds(b1 + o, 16)]
                                      + w2_v[pl.ds(b2 + o, 16)])
        return carry

    lax.fori_loop(0, T_ROWS, build, 0)

    iota = lax.iota(jnp.int32, 16)

    def chunk_body(k, carry):
        cid = wid + k * NW
        base = cid * C

        pltpu.sync_copy(ea_hbm.at[pl.ds(base * 3, C * 3)], attr_v)

        @plsc.parallel_loop(0, C // 16)
        def group(g):
            row = g * 16 + iota
            r3 = row * 3
            a0 = plsc.load_gather(attr_v, [r3])
            a1 = plsc.load_gather(attr_v, [r3 + 1])
            a2 = plsc.load_gather(attr_v, [r3 + 2])
            a0 = jnp.minimum(jnp.maximum(a0, 0), 4)
            a1 = jnp.minimum(jnp.maximum(a1, 0), 5)
            a2 = jnp.minimum(jnp.maximum(a2, 0), 1)
            c = a0 * 12 + a1 * 2 + a2
            tb = c * D

            @plsc.parallel_loop(0, D, unroll=8)
            def jloop(j):
                col = (iota + j) & (D - 1)
                v = plsc.load_gather(t_v, [tb + col])
                plsc.store_scatter(out_v, [row, col], v)
        pltpu.sync_copy(out_v, out_hbm.at[pl.ds(base, C)])
        return carry

    nchunks_w = (num_chunks - wid + NW - 1) // NW
    lax.fori_loop(0, nchunks_w, chunk_body, 0)


def kernel(edge_attr, W_bond_type, W_bond_stereo, W_is_conjugated):
    n_edges = edge_attr.shape[0]
    mesh = plsc.VectorSubcoreMesh(core_axis_name="c", subcore_axis_name="s")
    enc = functools.partial(
        pl.kernel,
        mesh=mesh,
        compiler_params=pltpu.CompilerParams(needs_layout_passes=False),
        out_type=jax.ShapeDtypeStruct((n_edges, D), jnp.float32),
        scratch_types=[
            pltpu.VMEM((C * 3,), jnp.int32),
            pltpu.VMEM((C, D), jnp.float32),
            pltpu.VMEM((5 * D,), jnp.float32),
            pltpu.VMEM((6 * D,), jnp.float32),
            pltpu.VMEM((2 * D,), jnp.float32),
            pltpu.VMEM((T_ROWS * D,), jnp.float32),
        ],
    )(functools.partial(_encoder_body, n_edges))
    return enc(edge_attr.reshape(-1),
               W_bond_type.reshape(-1),
               W_bond_stereo.reshape(-1),
               W_is_conjugated.reshape(-1))


# speedup vs baseline: 1.7814x; 1.7814x over previous
import functools

import jax
import jax.numpy as jnp
from jax import lax
from jax.experimental import pallas as pl
from jax.experimental.pallas import tpu as pltpu
from jax.experimental.pallas import tpu_sc as plsc

D = 128
C = 512
NW = 32
T_ROWS = 60


def _encoder_body(n_edges, ea0_hbm, ea1_hbm, ea2_hbm, w0_hbm, w1_hbm, w2_hbm,
                  out_hbm, a0_v, a1_v, a2_v, out_v, w0_v, w1_v, w2_v, t_v):
    num_chunks = n_edges // C
    wid = lax.axis_index("s") * 2 + lax.axis_index("c")

    pltpu.sync_copy(w0_hbm, w0_v)
    pltpu.sync_copy(w1_hbm, w1_v)
    pltpu.sync_copy(w2_hbm, w2_v)

    def build(c60, carry):
        i0 = c60 // 12
        r = c60 - i0 * 12
        i1 = r // 2
        i2 = r - i1 * 2
        tb = pl.multiple_of(c60 * D, D)
        b0 = pl.multiple_of(i0 * D, D)
        b1 = pl.multiple_of(i1 * D, D)
        b2 = pl.multiple_of(i2 * D, D)
        for j in range(D // 16):
            o = j * 16
            t_v[pl.ds(tb + o, 16)] = (w0_v[pl.ds(b0 + o, 16)]
                                      + w1_v[pl.ds(b1 + o, 16)]
                                      + w2_v[pl.ds(b2 + o, 16)])
        return carry

    lax.fori_loop(0, T_ROWS, build, 0)

    iota = lax.iota(jnp.int32, 16)

    def chunk_body(k, carry):
        cid = wid + k * NW
        base = cid * C

        pltpu.sync_copy(ea0_hbm.at[pl.ds(base, C)], a0_v)
        pltpu.sync_copy(ea1_hbm.at[pl.ds(base, C)], a1_v)
        pltpu.sync_copy(ea2_hbm.at[pl.ds(base, C)], a2_v)

        @plsc.parallel_loop(0, C // 16)
        def group(g):
            row = g * 16 + iota
            e16 = pl.multiple_of(g * 16, 16)
            a0 = a0_v[pl.ds(e16, 16)]
            a1 = a1_v[pl.ds(e16, 16)]
            a2 = a2_v[pl.ds(e16, 16)]
            a0 = jnp.minimum(jnp.maximum(a0, 0), 4)
            a1 = jnp.minimum(jnp.maximum(a1, 0), 5)
            a2 = jnp.minimum(jnp.maximum(a2, 0), 1)
            c = a0 * 12 + a1 * 2 + a2
            tb = c * D

            @plsc.parallel_loop(0, D, unroll=8)
            def jloop(j):
                col = (iota + j) & (D - 1)
                v = plsc.load_gather(t_v, [tb + col])
                plsc.store_scatter(out_v, [row, col], v)
        pltpu.sync_copy(out_v, out_hbm.at[pl.ds(base, C)])
        return carry

    nchunks_w = (num_chunks - wid + NW - 1) // NW
    lax.fori_loop(0, nchunks_w, chunk_body, 0)


def kernel(edge_attr, W_bond_type, W_bond_stereo, W_is_conjugated):
    n_edges = edge_attr.shape[0]
    mesh = plsc.VectorSubcoreMesh(core_axis_name="c", subcore_axis_name="s")
    enc = functools.partial(
        pl.kernel,
        mesh=mesh,
        compiler_params=pltpu.CompilerParams(needs_layout_passes=False),
        out_type=jax.ShapeDtypeStruct((n_edges, D), jnp.float32),
        scratch_types=[
            pltpu.VMEM((C,), jnp.int32),
            pltpu.VMEM((C,), jnp.int32),
            pltpu.VMEM((C,), jnp.int32),
            pltpu.VMEM((C, D), jnp.float32),
            pltpu.VMEM((5 * D,), jnp.float32),
            pltpu.VMEM((6 * D,), jnp.float32),
            pltpu.VMEM((2 * D,), jnp.float32),
            pltpu.VMEM((T_ROWS * D,), jnp.float32),
        ],
    )(functools.partial(_encoder_body, n_edges))
    return enc(edge_attr[:, 0], edge_attr[:, 1], edge_attr[:, 2],
               W_bond_type.reshape(-1),
               W_bond_stereo.reshape(-1),
               W_is_conjugated.reshape(-1))


# speedup vs baseline: 2.9578x; 1.6603x over previous
import functools

import jax
import jax.numpy as jnp
from jax import lax
from jax.experimental import pallas as pl
from jax.experimental.pallas import tpu as pltpu
from jax.experimental.pallas import tpu_sc as plsc

D = 128
C = 256
NW = 32
T_ROWS = 60


def _encoder_body(n_edges, ea0_hbm, ea1_hbm, ea2_hbm, w0_hbm, w1_hbm, w2_hbm,
                  out_hbm, a0_v, a1_v, a2_v, out_v, w0_v, w1_v, w2_v, t_v,
                  sem_in0, sem_in1, sem_out0, sem_out1):
    num_chunks = n_edges // C
    wid = lax.axis_index("s") * 2 + lax.axis_index("c")
    sems_in = (sem_in0, sem_in1)
    sems_out = (sem_out0, sem_out1)

    pltpu.sync_copy(w0_hbm, w0_v)
    pltpu.sync_copy(w1_hbm, w1_v)
    pltpu.sync_copy(w2_hbm, w2_v)

    def build(c60, carry):
        i0 = c60 // 12
        r = c60 - i0 * 12
        i1 = r // 2
        i2 = r - i1 * 2
        tb = pl.multiple_of(c60 * D, D)
        b0 = pl.multiple_of(i0 * D, D)
        b1 = pl.multiple_of(i1 * D, D)
        b2 = pl.multiple_of(i2 * D, D)
        for j in range(D // 16):
            o = j * 16
            t_v[pl.ds(tb + o, 16)] = (w0_v[pl.ds(b0 + o, 16)]
                                      + w1_v[pl.ds(b1 + o, 16)]
                                      + w2_v[pl.ds(b2 + o, 16)])
        return carry

    lax.fori_loop(0, T_ROWS, build, 0)

    iota = lax.iota(jnp.int32, 16)
    nw = (num_chunks - wid + NW - 1) // NW

    def issue_in(k, b):
        base = (wid + k * NW) * C
        pltpu.async_copy(ea0_hbm.at[pl.ds(base, C)],
                         a0_v.at[pl.ds(b * C, C)], sems_in[b])
        pltpu.async_copy(ea1_hbm.at[pl.ds(base, C)],
                         a1_v.at[pl.ds(b * C, C)], sems_in[b])
        pltpu.async_copy(ea2_hbm.at[pl.ds(base, C)],
                         a2_v.at[pl.ds(b * C, C)], sems_in[b])

    def drain_in(b):
        for a_v in (a0_v, a1_v, a2_v):
            pltpu.make_async_copy(ea0_hbm.at[pl.ds(0, C)],
                                  a_v.at[pl.ds(b * C, C)], sems_in[b]).wait()

    def drain_out(b):
        pltpu.make_async_copy(out_v.at[pl.ds(b * C, C)],
                              out_hbm.at[pl.ds(0, C)], sems_out[b]).wait()

    @pl.when(nw > 0)
    def _prime():
        issue_in(0, 0)

    def pair_body(h, carry):
        for b in (0, 1):
            k = 2 * h + b

            @pl.when(k < nw)
            def _chunk():
                base = (wid + k * NW) * C
                drain_in(b)

                @pl.when(k + 1 < nw)
                def _prefetch():
                    issue_in(k + 1, 1 - b)

                @pl.when(k >= 2)
                def _reclaim():
                    drain_out(b)

                rb = b * C

                @plsc.parallel_loop(0, C // 16)
                def group(g):
                    row = g * 16 + iota
                    e16 = pl.multiple_of(rb + g * 16, 16)
                    a0 = a0_v[pl.ds(e16, 16)]
                    a1 = a1_v[pl.ds(e16, 16)]
                    a2 = a2_v[pl.ds(e16, 16)]
                    a0 = jnp.minimum(jnp.maximum(a0, 0), 4)
                    a1 = jnp.minimum(jnp.maximum(a1, 0), 5)
                    a2 = jnp.minimum(jnp.maximum(a2, 0), 1)
                    c = a0 * 12 + a1 * 2 + a2
                    tb = c * D

                    @plsc.parallel_loop(0, D, unroll=8)
                    def jloop(j):
                        col = (iota + j) & (D - 1)
                        v = plsc.load_gather(t_v, [tb + col])
                        plsc.store_scatter(out_v, [rb + row, col], v)

                pltpu.async_copy(out_v.at[pl.ds(rb, C)],
                                 out_hbm.at[pl.ds(base, C)], sems_out[b])
        return carry

    lax.fori_loop(0, (nw + 1) // 2, pair_body, 0)

    @pl.when(nw > 0)
    def _tail0():
        drain_out(0)

    @pl.when(nw > 1)
    def _tail1():
        drain_out(1)


def kernel(edge_attr, W_bond_type, W_bond_stereo, W_is_conjugated):
    n_edges = edge_attr.shape[0]
    mesh = plsc.VectorSubcoreMesh(core_axis_name="c", subcore_axis_name="s")
    enc = functools.partial(
        pl.kernel,
        mesh=mesh,
        compiler_params=pltpu.CompilerParams(needs_layout_passes=False),
        out_type=jax.ShapeDtypeStruct((n_edges, D), jnp.float32),
        scratch_types=[
            pltpu.VMEM((2 * C,), jnp.int32),
            pltpu.VMEM((2 * C,), jnp.int32),
            pltpu.VMEM((2 * C,), jnp.int32),
            pltpu.VMEM((2 * C, D), jnp.float32),
            pltpu.VMEM((5 * D,), jnp.float32),
            pltpu.VMEM((6 * D,), jnp.float32),
            pltpu.VMEM((2 * D,), jnp.float32),
            pltpu.VMEM((T_ROWS * D,), jnp.float32),
            pltpu.SemaphoreType.DMA,
            pltpu.SemaphoreType.DMA,
            pltpu.SemaphoreType.DMA,
            pltpu.SemaphoreType.DMA,
        ],
    )(functools.partial(_encoder_body, n_edges))
    return enc(edge_attr[:, 0], edge_attr[:, 1], edge_attr[:, 2],
               W_bond_type.reshape(-1),
               W_bond_stereo.reshape(-1),
               W_is_conjugated.reshape(-1))
